# baseline (speedup 1.0000x reference)
import functools

import jax
import jax.numpy as jnp
from jax import lax
from jax.experimental import pallas as pl
from jax.experimental.pallas import tpu as pltpu
from jax.experimental.pallas import tpu_sc as plsc

_EMBED = 128
_BATCH = 32
_SEQ = 8192
_B = _BATCH * _SEQ

_NC = 2
_NS = 16
_NW = _NC * _NS
_PER_W = _B // _NW
_CHUNK = 128
_NIDX = _PER_W // _CHUNK
_RPC = 2
_CW = _RPC * _CHUNK
_NCHUNK = _PER_W // _CW
_NBUF = 2

_gather_rows_cache = None


def _build():
    global _gather_rows_cache
    if _gather_rows_cache is not None:
        return _gather_rows_cache

    mesh = plsc.VectorSubcoreMesh(core_axis_name="c", subcore_axis_name="s")

    @functools.partial(
        pl.kernel,
        mesh=mesh,
        out_type=jax.ShapeDtypeStruct((_B, _EMBED), jnp.float32),
        scratch_types=(
            [pltpu.VMEM((_NIDX, _CHUNK), jnp.int32)]
            + [pltpu.VMEM((_CW, _EMBED), jnp.float32)] * _NBUF
            + [pltpu.SemaphoreType.DMA] * (2 * _NBUF)
        ),
    )
    def _gather_rows(table_hbm, idx_hbm, out_hbm, idx_v, *scr):
        bufs = scr[:_NBUF]
        gsems = scr[_NBUF:2 * _NBUF]
        wsems = scr[2 * _NBUF:]
        wid = lax.axis_index("s") * _NC + lax.axis_index("c")
        ibase = wid * _NIDX

        def start_gathers(cur, b):
            for h in range(_RPC):
                pltpu.async_copy(
                    table_hbm.at[idx_v.at[_RPC * cur + h]],
                    bufs[b].at[pl.ds(h * _CHUNK, _CHUNK)], gsems[b])

        def wait_gathers(cur, b):
            for h in range(_RPC):
                pltpu.make_async_copy(
                    table_hbm.at[idx_v.at[_RPC * cur + h]],
                    bufs[b].at[pl.ds(h * _CHUNK, _CHUNK)], gsems[b]).wait()

        def write_out(cur, b):
            out_slice = out_hbm.at[pl.ds((ibase * _CHUNK) + cur * _CW, _CW)]
            pltpu.async_copy(bufs[b], out_slice, wsems[b])
            pltpu.make_async_copy(bufs[b], out_slice, wsems[b]).wait()

        pltpu.sync_copy(idx_hbm.at[pl.ds(ibase, _NIDX)], idx_v)

        for b in range(_NBUF):
            start_gathers(b, b)

        def body(j, carry):
            for b in range(_NBUF):
                cur = _NBUF * j + b
                wait_gathers(cur, b)
                write_out(cur, b)
                start_gathers(cur + _NBUF, b)
            return carry

        lax.fori_loop(0, (_NCHUNK - _NBUF) // _NBUF, body, 0)

        for b in range(_NBUF):
            cur = _NCHUNK - _NBUF + b
            wait_gathers(cur, b)
            write_out(cur, b)

    _gather_rows_cache = _gather_rows
    return _gather_rows


def kernel(position, pe):
    idx = position.reshape(_B // _CHUNK, _CHUNK)
    out = _build()(pe, idx)
    return out.reshape(_BATCH, _SEQ, _EMBED)
